# baseline (speedup 1.0000x reference)
import functools

import jax
import jax.numpy as jnp
from jax import lax
from jax.experimental import pallas as pl
from jax.experimental.pallas import tpu as pltpu
from jax.experimental.pallas import tpu_sc as plsc

B, C, T = 128, 2, 122880
T_OUT = 23552
TH, LH = 960, 128
OH = 184
ROW = OH * C * LH
BSTRIDE = TH * C * LH
NC, NS = 2, 16
NW = NC * NS
BPW = B // NW

_mesh = plsc.VectorSubcoreMesh(core_axis_name="c", subcore_axis_name="s")


def _base_indices():
    o = jnp.arange(T_OUT, dtype=jnp.int32)
    t = jnp.where(
        o < 10240,
        o * 8,
        jnp.where(o < 19456, 81920 + (o - 10240) * 4, 118784 + (o - 19456)),
    )
    addr = (t >> 7) * (C * LH) + (t & 127)
    addr2 = jnp.stack([addr, addr + LH], axis=0)
    return addr2.reshape(C, OH, LH).transpose(1, 0, 2).reshape(ROW)


@functools.partial(
    pl.kernel,
    out_type=jax.ShapeDtypeStruct((B, ROW), jnp.float32),
    mesh=_mesh,
    scratch_types=[
        pltpu.VMEM((ROW,), jnp.int32),
        pltpu.VMEM((ROW,), jnp.float32),
        pltpu.SemaphoreType.DMA,
        pltpu.SemaphoreType.DMA,
        pltpu.SemaphoreType.DMA,
    ],
)
def _decimate(strain_hbm, idx_hbm, out_hbm, idx_v, row_v, sg, so, si):
    widx = lax.axis_index("s") * NC + lax.axis_index("c")
    b0 = widx * BPW

    pltpu.async_copy(idx_hbm, idx_v, si).wait()
    off0 = b0 * BSTRIDE

    def add_off(i, off):
        for u in range(4):
            s = (i * 4 + u) * 16
            idx_v[pl.ds(s, 16)] = idx_v[pl.ds(s, 16)] + off
        return off

    lax.fori_loop(0, ROW // 64, add_off, off0)

    out_cp = None
    for b_local in range(BPW):
        b = b0 + b_local
        if out_cp is not None:
            out_cp.wait()
        pltpu.async_copy(strain_hbm.at[idx_v], row_v, sg).wait()
        out_cp = pltpu.async_copy(row_v, out_hbm.at[b], so)
        if b_local + 1 < BPW:
            lax.fori_loop(0, ROW // 64, add_off, jnp.int32(BSTRIDE))

    out_cp.wait()


def kernel(strain):
    flat = strain.reshape(B, C, TH, LH).transpose(0, 2, 1, 3).reshape(B * TH * C * LH)
    y = _decimate(flat, _base_indices())
    return (
        y.reshape(B, OH, C, LH).transpose(0, 2, 1, 3).reshape(B, C, T_OUT)
    )


# speedup vs baseline: 3.4448x; 3.4448x over previous
import functools

import jax
import jax.numpy as jnp
from jax import lax
from jax.experimental import pallas as pl
from jax.experimental.pallas import tpu as pltpu
from jax.experimental.pallas import tpu_sc as plsc

B, C, T = 128, 2, 122880
T_OUT = 23552
TH, LH = 960, 128
OH = 184
NC, NS = 2, 16
NW = NC * NS
BPW = B // NW

JOBS = tuple(
    [(128 * k, 128, 8, 2048 * k) for k in range(5)]
    + [(640 + 96 * k, 96, 4, 10240 + 3072 * k) for k in range(3)]
)
NJ = len(JOBS)
S3_TH, S3_CNT, S3_OH = 928, 32, 152

IN_TH = 128

_mesh = plsc.VectorSubcoreMesh(core_axis_name="c", subcore_axis_name="s")


@functools.partial(
    pl.kernel,
    out_type=jax.ShapeDtypeStruct((B, OH, C, LH), jnp.float32),
    mesh=_mesh,
    scratch_types=[
        pltpu.VMEM((2, IN_TH, C, LH), jnp.float32),
        pltpu.VMEM((OH, C, LH), jnp.float32),
        pltpu.SemaphoreType.DMA,
        pltpu.SemaphoreType.DMA,
        pltpu.SemaphoreType.DMA,
        pltpu.SemaphoreType.DMA,
    ],
    compiler_params=pltpu.CompilerParams(
        needs_layout_passes=False,
        use_tc_tiling_on_sc=False,
    ),
)
def _decimate(strain_hbm, out_hbm, in_v, row_v, si0, si1, so, s3):
    widx = lax.axis_index("s") * NC + lax.axis_index("c")
    iota = lax.iota(jnp.int32, 16)
    tl8 = iota * 8
    tl4a = iota * 4
    tl4b = iota * 4 + 64
    sin = (si0, si1)

    def start_in(b, j, buf):
        th0, cnt, _, _ = JOBS[j]
        return pltpu.async_copy(
            strain_hbm.at[b, pl.ds(th0, cnt)],
            in_v.at[buf, pl.ds(0, cnt)],
            sin[buf],
        )

    def dec_chunk(j, buf):
        _, cnt, stride, o0 = JOBS[j]
        for c in range(C):
            cvec = iota * 0 + c
            if stride == 8:
                def dec8(i, cr, buf=buf, c=c, cvec=cvec, o0=o0):
                    for u in range(4):
                        th = i * 4 + u
                        g = plsc.load_gather(
                            in_v.at[buf], [iota * 0 + th, cvec, tl8]
                        )
                        o = o0 + th * 16
                        row_v[o >> 7, c, pl.ds(o & 127, 16)] = g
                    return cr

                lax.fori_loop(0, cnt // 4, dec8, 0)
            else:
                def dec4(i, cr, buf=buf, c=c, cvec=cvec, o0=o0):
                    for u in range(2):
                        th = i * 2 + u
                        thv = iota * 0 + th
                        o = o0 + th * 32
                        g = plsc.load_gather(in_v.at[buf], [thv, cvec, tl4a])
                        row_v[o >> 7, c, pl.ds(o & 127, 16)] = g
                        g = plsc.load_gather(in_v.at[buf], [thv, cvec, tl4b])
                        o = o + 16
                        row_v[o >> 7, c, pl.ds(o & 127, 16)] = g
                    return cr

                lax.fori_loop(0, cnt // 2, dec4, 0)

    out_cp = None
    for b_local in range(BPW):
        b = widx * BPW + b_local

        cps = [None] * NJ
        cps[0] = start_in(b, 0, 0)
        cps[1] = start_in(b, 1, 1)

        if out_cp is not None:
            out_cp.wait()

        c3 = pltpu.async_copy(
            strain_hbm.at[b, pl.ds(S3_TH, S3_CNT)],
            row_v.at[pl.ds(S3_OH, S3_CNT)],
            s3,
        )

        for j in range(NJ):
            buf = j & 1
            cps[j].wait()
            dec_chunk(j, buf)
            if j + 2 < NJ:
                cps[j + 2] = start_in(b, j + 2, buf)

        c3.wait()
        out_cp = pltpu.async_copy(row_v, out_hbm.at[b], so)

    out_cp.wait()


def kernel(strain):
    a = strain.reshape(B, C, TH, LH).transpose(0, 2, 1, 3)
    y = _decimate(a)
    return y.transpose(0, 2, 1, 3).reshape(B, C, T_OUT)
